# Initial kernel scaffold; baseline (speedup 1.0000x reference)
import jax
import jax.numpy as jnp
from jax.experimental import pallas as pl


def kernel(code, table):
    raise NotImplementedError("write your pallas kernel here")


# speedup vs baseline: 2.7080x; 2.7080x over previous
import functools

import jax
import jax.numpy as jnp
from jax import lax
from jax.experimental import pallas as pl
from jax.experimental.pallas import tpu as pltpu
from jax.experimental.pallas import tpu_sc as plsc

_REPEATS = 2
_B, _L, _D = 1024, 200, 32
_N = _B * _L

_NC, _NS = 2, 16
_NW = _NC * _NS
_N_PER_W = _N // _NW
_CHUNK = 640
_N_CHUNKS = _N_PER_W // _CHUNK


def _sc_body(code_hbm, table_hbm, out_hbm, idx_v, rows_v, sem):
    c = lax.axis_index("c")
    s = lax.axis_index("s")
    wid = s * _NC + c
    base = wid * _N_PER_W

    def chunk_body(i, carry):
        off = base + i * _CHUNK
        pltpu.sync_copy(code_hbm.at[pl.ds(off, _CHUNK)], idx_v)
        pltpu.async_copy(table_hbm.at[idx_v], rows_v, sem).wait()
        pltpu.sync_copy(rows_v, out_hbm.at[pl.ds(off, _CHUNK), 0])
        pltpu.sync_copy(rows_v, out_hbm.at[pl.ds(off, _CHUNK), 1])
        return carry

    lax.fori_loop(0, _N_CHUNKS, chunk_body, 0)


_mesh = plsc.VectorSubcoreMesh(
    core_axis_name="c", subcore_axis_name="s", num_cores=_NC, num_subcores=_NS
)

_sc_call = pl.kernel(
    _sc_body,
    out_type=jax.ShapeDtypeStruct((_N, _REPEATS, _D), jnp.float32),
    mesh=_mesh,
    scratch_types=[
        pltpu.VMEM((_CHUNK,), jnp.int32),
        pltpu.VMEM((_CHUNK, _D), jnp.float32),
        pltpu.SemaphoreType.DMA,
    ],
    compiler_params=pltpu.CompilerParams(use_tc_tiling_on_sc=False),
)


@jax.jit
def kernel(code, table):
    code_flat = code.reshape(-1).astype(jnp.int32)
    out = _sc_call(code_flat, table)
    return out.reshape(_B, _L * _REPEATS, _D)


# speedup vs baseline: 2.7902x; 1.0304x over previous
import functools

import jax
import jax.numpy as jnp
from jax import lax
from jax.experimental import pallas as pl
from jax.experimental.pallas import tpu as pltpu
from jax.experimental.pallas import tpu_sc as plsc

_REPEATS = 2
_B, _L, _D = 1024, 200, 32
_N = _B * _L

_NC, _NS = 2, 16
_NW = _NC * _NS
_N_PER_W = _N // _NW
_CHUNK = 800
_N_CHUNKS = _N_PER_W // _CHUNK
_NBUF = 2


def _sc_body(code_hbm, table_hbm, out_hbm, idx_v, rows_v, sem_g, sem_w):
    c = lax.axis_index("c")
    s = lax.axis_index("s")
    wid = s * _NC + c
    base = wid * _N_PER_W

    pltpu.sync_copy(code_hbm.at[pl.ds(base, _N_PER_W)], idx_v)

    def gather(i, b):
        return pltpu.async_copy(
            table_hbm.at[idx_v.at[pl.ds(i * _CHUNK, _CHUNK)]], rows_v.at[b], sem_g
        )

    def writes(i, b):
        off = base + i * _CHUNK
        w0 = pltpu.async_copy(rows_v.at[b], out_hbm.at[pl.ds(off, _CHUNK), 0], sem_w)
        w1 = pltpu.async_copy(rows_v.at[b], out_hbm.at[pl.ds(off, _CHUNK), 1], sem_w)
        return (w0, w1)

    g = gather(0, 0)
    pending = None
    for i in range(_N_CHUNKS):
        g.wait()
        if i + 1 < _N_CHUNKS:
            if pending is not None:
                for w in pending:
                    w.wait()
            g = gather(i + 1, (i + 1) % _NBUF)
        new_pending = writes(i, i % _NBUF)
        if i + 1 >= _N_CHUNKS and pending is not None:
            for w in pending:
                w.wait()
        pending = new_pending
    for w in pending:
        w.wait()


_mesh = plsc.VectorSubcoreMesh(
    core_axis_name="c", subcore_axis_name="s", num_cores=_NC, num_subcores=_NS
)

_sc_call = pl.kernel(
    _sc_body,
    out_type=jax.ShapeDtypeStruct((_N, _REPEATS, _D), jnp.float32),
    mesh=_mesh,
    scratch_types=[
        pltpu.VMEM((_N_PER_W,), jnp.int32),
        pltpu.VMEM((_NBUF, _CHUNK, _D), jnp.float32),
        pltpu.SemaphoreType.DMA,
        pltpu.SemaphoreType.DMA,
    ],
    compiler_params=pltpu.CompilerParams(use_tc_tiling_on_sc=False),
)


@jax.jit
def kernel(code, table):
    code_flat = code.reshape(-1).astype(jnp.int32)
    out = _sc_call(code_flat, table)
    return out.reshape(_B, _L * _REPEATS, _D)


# speedup vs baseline: 3.4607x; 1.2403x over previous
import functools

import jax
import jax.numpy as jnp
from jax import lax
from jax.experimental import pallas as pl
from jax.experimental.pallas import tpu as pltpu
from jax.experimental.pallas import tpu_sc as plsc

_REPEATS = 2
_B, _L, _D = 1024, 200, 32
_N = _B * _L

_NC, _NS = 2, 16
_NW = _NC * _NS
_BB = 128
_NBT = _B // _BB
_TASKS = _L * _NBT
_TPW = _TASKS // _NW
_NPW = _TPW * _BB


def _sc_body(
    ct_hbm, table_hbm, out_hbm,
    idx_v, rows_a, rows_b, trans_a, trans_b,
    sem_a, sem_b, wsem_a, wsem_b,
):
    c = lax.axis_index("c")
    s = lax.axis_index("s")
    wid = s * _NC + c
    base_t = wid * _TPW

    pltpu.sync_copy(ct_hbm.at[pl.ds(wid * _NPW, _NPW)], idx_v)

    iota16 = lax.iota(jnp.int32, 16)
    row_idx = [iota16 + (b1 * 16) for b1 in range(_BB // 16)]

    def gather(i, rows, sem):
        return pltpu.async_copy(
            table_hbm.at[idx_v.at[pl.ds(i * _BB, _BB)]], rows, sem
        )

    def wait_gather(rows, sem):
        pltpu.make_async_copy(table_hbm.at[pl.ds(0, _BB)], rows, sem).wait()

    def transpose(rows, trans):
        for d in range(_D):
            col = jnp.full((16,), d, jnp.int32)
            for b1 in range(_BB // 16):
                vec = plsc.load_gather(rows, [row_idx[b1], col])
                trans[d // 8, d % 8, pl.ds(b1 * 16, 16)] = vec

    def writes(i, trans, wsem):
        t = base_t + i
        lp = t // _NBT
        bt = t % _NBT
        pltpu.async_copy(trans, out_hbm.at[2 * lp, :, bt], wsem)
        pltpu.async_copy(trans, out_hbm.at[2 * lp + 1, :, bt], wsem)

    def wait_writes(trans, wsem):
        pltpu.make_async_copy(trans, out_hbm.at[0, :, 0], wsem).wait()
        pltpu.make_async_copy(trans, out_hbm.at[0, :, 0], wsem).wait()

    gather(0, rows_a, sem_a)

    def body(j, carry):
        t0 = 2 * j
        wait_gather(rows_a, sem_a)
        gather(t0 + 1, rows_b, sem_b)

        @pl.when(j > 0)
        def _():
            wait_writes(trans_a, wsem_a)

        transpose(rows_a, trans_a)
        writes(t0, trans_a, wsem_a)

        wait_gather(rows_b, sem_b)

        @pl.when(j < _TPW // 2 - 1)
        def _():
            gather(t0 + 2, rows_a, sem_a)

        @pl.when(j > 0)
        def _():
            wait_writes(trans_b, wsem_b)

        transpose(rows_b, trans_b)
        writes(t0 + 1, trans_b, wsem_b)
        return carry

    lax.fori_loop(0, _TPW // 2, body, 0)
    wait_writes(trans_a, wsem_a)
    wait_writes(trans_b, wsem_b)


_mesh = plsc.VectorSubcoreMesh(
    core_axis_name="c", subcore_axis_name="s", num_cores=_NC, num_subcores=_NS
)

_sc_call = pl.kernel(
    _sc_body,
    out_type=jax.ShapeDtypeStruct(
        (_L * _REPEATS, _D // 8, _NBT, 8, _BB), jnp.float32
    ),
    mesh=_mesh,
    scratch_types=[
        pltpu.VMEM((_NPW,), jnp.int32),
        pltpu.VMEM((_BB, _D), jnp.float32),
        pltpu.VMEM((_BB, _D), jnp.float32),
        pltpu.VMEM((_D // 8, 8, _BB), jnp.float32),
        pltpu.VMEM((_D // 8, 8, _BB), jnp.float32),
        pltpu.SemaphoreType.DMA,
        pltpu.SemaphoreType.DMA,
        pltpu.SemaphoreType.DMA,
        pltpu.SemaphoreType.DMA,
    ],
    compiler_params=pltpu.CompilerParams(
        use_tc_tiling_on_sc=False, needs_layout_passes=False
    ),
)


@jax.jit
def kernel(code, table):
    ct_flat = code.T.reshape(-1).astype(jnp.int32)
    out5 = _sc_call(ct_flat, table)
    return out5.transpose((2, 4, 0, 1, 3)).reshape(_B, _L * _REPEATS, _D)


# speedup vs baseline: 7.0462x; 2.0360x over previous
import functools

import jax
import jax.numpy as jnp
from jax import lax
from jax.experimental import pallas as pl
from jax.experimental.pallas import tpu as pltpu
from jax.experimental.pallas import tpu_sc as plsc

_REPEATS = 2
_B, _L, _D = 1024, 200, 32
_N = _B * _L

_NC, _NS = 2, 16
_NW = _NC * _NS
_BB = 128
_NBT = _B // _BB
_TB = 1
_TR = _TB * _BB
_TASKS = _N // _TR
_TPW = _TASKS // _NW
_NPW = _TPW * _TR


def _sc_body(
    ct_hbm, table_hbm, out_hbm,
    idx_v, rows_a, rows_b, trans_a, trans_b,
    sem_a, sem_b, wsem_a, wsem_b,
):
    c = lax.axis_index("c")
    s = lax.axis_index("s")
    wid = s * _NC + c
    base_t = wid * _TPW

    pltpu.sync_copy(ct_hbm.at[pl.ds(wid * _NPW, _NPW)], idx_v)

    iota16 = lax.iota(jnp.int32, 16)
    dhi0 = iota16 // 8
    dhi1 = dhi0 + 2
    dlo_c = iota16 - dhi0 * 8
    zeros16 = jnp.zeros((16,), jnp.int32)

    def gather(i, rows, sem):
        return pltpu.async_copy(
            table_hbm.at[idx_v.at[pl.ds(i * _TR, _TR)]], rows, sem
        )

    def wait_gather(rows, sem):
        pltpu.make_async_copy(table_hbm.at[pl.ds(0, _TR)], rows, sem).wait()

    def transpose(rows, trans):
        @plsc.parallel_loop(0, _TR, unroll=8)
        def _(k):
            btl = k // _BB
            jcol = k - btl * _BB
            bv = zeros16 + btl
            jv = zeros16 + jcol
            vec0 = rows[k, pl.ds(0, 16)]
            vec1 = rows[k, pl.ds(16, 16)]
            plsc.store_scatter(trans, [dhi0, bv, dlo_c, jv], vec0)
            plsc.store_scatter(trans, [dhi1, bv, dlo_c, jv], vec1)

    def writes(i, trans, wsem):
        t = base_t + i
        lp = t // (_NBT // _TB)
        bt0 = (t % (_NBT // _TB)) * _TB
        src = trans.at[:, :, :, pl.ds(0, _BB)]
        pltpu.async_copy(src, out_hbm.at[2 * lp, :, pl.ds(bt0, _TB)], wsem)
        pltpu.async_copy(src, out_hbm.at[2 * lp + 1, :, pl.ds(bt0, _TB)], wsem)

    def wait_writes(trans, wsem):
        src = trans.at[:, :, :, pl.ds(0, _BB)]
        pltpu.make_async_copy(src, out_hbm.at[0, :, pl.ds(0, _TB)], wsem).wait()
        pltpu.make_async_copy(src, out_hbm.at[0, :, pl.ds(0, _TB)], wsem).wait()

    gather(0, rows_a, sem_a)

    def body(j, carry):
        t0 = 2 * j
        wait_gather(rows_a, sem_a)
        gather(t0 + 1, rows_b, sem_b)

        @pl.when(j > 0)
        def _():
            wait_writes(trans_a, wsem_a)

        transpose(rows_a, trans_a)
        writes(t0, trans_a, wsem_a)

        wait_gather(rows_b, sem_b)

        @pl.when(t0 + 2 < _TPW)
        def _():
            gather(t0 + 2, rows_a, sem_a)

        @pl.when(j > 0)
        def _():
            wait_writes(trans_b, wsem_b)

        transpose(rows_b, trans_b)
        writes(t0 + 1, trans_b, wsem_b)
        return carry

    lax.fori_loop(0, _TPW // 2, body, 0)

    if _TPW % 2 == 1:
        wait_gather(rows_a, sem_a)
        wait_writes(trans_a, wsem_a)
        transpose(rows_a, trans_a)
        writes(_TPW - 1, trans_a, wsem_a)
    wait_writes(trans_b, wsem_b)
    wait_writes(trans_a, wsem_a)


_mesh = plsc.VectorSubcoreMesh(
    core_axis_name="c", subcore_axis_name="s", num_cores=_NC, num_subcores=_NS
)

_sc_call = pl.kernel(
    _sc_body,
    out_type=jax.ShapeDtypeStruct(
        (_L * _REPEATS, _D // 8, _NBT, 8, _BB), jnp.float32
    ),
    mesh=_mesh,
    scratch_types=[
        pltpu.VMEM((_NPW,), jnp.int32),
        pltpu.VMEM((_TR, _D), jnp.float32),
        pltpu.VMEM((_TR, _D), jnp.float32),
        pltpu.VMEM((_D // 8, _TB, 8, _BB + 1), jnp.float32),
        pltpu.VMEM((_D // 8, _TB, 8, _BB + 1), jnp.float32),
        pltpu.SemaphoreType.DMA,
        pltpu.SemaphoreType.DMA,
        pltpu.SemaphoreType.DMA,
        pltpu.SemaphoreType.DMA,
    ],
    compiler_params=pltpu.CompilerParams(
        use_tc_tiling_on_sc=False, needs_layout_passes=False
    ),
)


@jax.jit
def kernel(code, table):
    ct_flat = code.T.reshape(-1).astype(jnp.int32)
    out5 = _sc_call(ct_flat, table)
    return out5.transpose((2, 4, 0, 1, 3)).reshape(_B, _L * _REPEATS, _D)


# speedup vs baseline: 8.0653x; 1.1446x over previous
import functools

import jax
import jax.numpy as jnp
from jax import lax
from jax.experimental import pallas as pl
from jax.experimental.pallas import tpu as pltpu
from jax.experimental.pallas import tpu_sc as plsc

_REPEATS = 2
_B, _L, _D = 1024, 200, 32
_N = _B * _L

_NC, _NS = 2, 16
_NW = _NC * _NS
_BB = 128
_NBT = _B // _BB
_TB = 2
_TR = _TB * _BB
_TASKS = _N // _TR
_TPW = _TASKS // _NW
_NPW = _TPW * _TR


def _sc_body(
    ct_hbm, table_hbm, out_hbm,
    idx_v, rows_a, rows_b, trans_a, trans_b,
    sem_a, sem_b, wsem_a, wsem_b,
):
    c = lax.axis_index("c")
    s = lax.axis_index("s")
    wid = s * _NC + c
    base_t = wid * _TPW

    pltpu.sync_copy(ct_hbm.at[pl.ds(wid * _NPW, _NPW)], idx_v)

    iota16 = lax.iota(jnp.int32, 16)
    dhi0 = iota16 // 8
    dhi1 = dhi0 + 2
    dlo_c = iota16 - dhi0 * 8
    zeros16 = jnp.zeros((16,), jnp.int32)

    def gather(i, rows, sem):
        return pltpu.async_copy(
            table_hbm.at[idx_v.at[pl.ds(i * _TR, _TR)]], rows, sem
        )

    def wait_gather(rows, sem):
        pltpu.make_async_copy(table_hbm.at[pl.ds(0, _TR)], rows, sem).wait()

    def transpose(rows, trans):
        @plsc.parallel_loop(0, _TR, unroll=8)
        def _(k):
            btl = k // _BB
            jcol = k - btl * _BB
            bv = zeros16 + btl
            jv = zeros16 + jcol
            vec0 = rows[k, pl.ds(0, 16)]
            vec1 = rows[k, pl.ds(16, 16)]
            plsc.store_scatter(trans, [bv, dhi0, dlo_c, jv], vec0)
            plsc.store_scatter(trans, [bv, dhi1, dlo_c, jv], vec1)

    def writes(i, trans, wsem):
        t = base_t + i
        lp = t // (_NBT // _TB)
        bt0 = (t % (_NBT // _TB)) * _TB
        for btl in range(_TB):
            src = trans.at[btl, :, :, pl.ds(0, _BB)]
            pltpu.async_copy(src, out_hbm.at[2 * lp, :, bt0 + btl], wsem)
            pltpu.async_copy(src, out_hbm.at[2 * lp + 1, :, bt0 + btl], wsem)

    def wait_writes(trans, wsem):
        for _ in range(2 * _TB):
            pltpu.make_async_copy(
                trans.at[0, :, :, pl.ds(0, _BB)], out_hbm.at[0, :, 0], wsem
            ).wait()

    gather(0, rows_a, sem_a)

    def body(j, carry):
        t0 = 2 * j
        wait_gather(rows_a, sem_a)
        gather(t0 + 1, rows_b, sem_b)

        @pl.when(j > 0)
        def _():
            wait_writes(trans_a, wsem_a)

        transpose(rows_a, trans_a)
        writes(t0, trans_a, wsem_a)

        wait_gather(rows_b, sem_b)

        @pl.when(t0 + 2 < _TPW)
        def _():
            gather(t0 + 2, rows_a, sem_a)

        @pl.when(j > 0)
        def _():
            wait_writes(trans_b, wsem_b)

        transpose(rows_b, trans_b)
        writes(t0 + 1, trans_b, wsem_b)
        return carry

    lax.fori_loop(0, _TPW // 2, body, 0)

    if _TPW % 2 == 1:
        wait_gather(rows_a, sem_a)
        wait_writes(trans_a, wsem_a)
        transpose(rows_a, trans_a)
        writes(_TPW - 1, trans_a, wsem_a)
    wait_writes(trans_b, wsem_b)
    wait_writes(trans_a, wsem_a)


_mesh = plsc.VectorSubcoreMesh(
    core_axis_name="c", subcore_axis_name="s", num_cores=_NC, num_subcores=_NS
)

_sc_call = pl.kernel(
    _sc_body,
    out_type=jax.ShapeDtypeStruct(
        (_L * _REPEATS, _D // 8, _NBT, 8, _BB), jnp.float32
    ),
    mesh=_mesh,
    scratch_types=[
        pltpu.VMEM((_NPW,), jnp.int32),
        pltpu.VMEM((_TR, _D), jnp.float32),
        pltpu.VMEM((_TR, _D), jnp.float32),
        pltpu.VMEM((_TB, _D // 8, 8, _BB + 1), jnp.float32),
        pltpu.VMEM((_TB, _D // 8, 8, _BB + 1), jnp.float32),
        pltpu.SemaphoreType.DMA,
        pltpu.SemaphoreType.DMA,
        pltpu.SemaphoreType.DMA,
        pltpu.SemaphoreType.DMA,
    ],
    compiler_params=pltpu.CompilerParams(
        use_tc_tiling_on_sc=False, needs_layout_passes=False
    ),
)


@jax.jit
def kernel(code, table):
    ct_flat = code.T.reshape(-1).astype(jnp.int32)
    out5 = _sc_call(ct_flat, table)
    return out5.transpose((2, 4, 0, 1, 3)).reshape(_B, _L * _REPEATS, _D)


# speedup vs baseline: 8.5527x; 1.0604x over previous
import functools

import jax
import jax.numpy as jnp
from jax import lax
from jax.experimental import pallas as pl
from jax.experimental.pallas import tpu as pltpu
from jax.experimental.pallas import tpu_sc as plsc

_REPEATS = 2
_B, _L, _D = 1024, 200, 32
_N = _B * _L

_NC, _NS = 2, 16
_NW = _NC * _NS
_BB = 128
_NBT = _B // _BB
_TB = 2
_TR = _TB * _BB
_TASKS = _N // _TR
_TPW = _TASKS // _NW
_NPW = _TPW * _TR


def _sc_body(
    ct_hbm, table_hbm, out_hbm,
    idx_v, rows_a, rows_b, trans_a, trans_b,
    sem_a, sem_b, wsem_a, wsem_b,
):
    c = lax.axis_index("c")
    s = lax.axis_index("s")
    wid = s * _NC + c
    base_t = wid * _TPW

    pltpu.sync_copy(ct_hbm.at[pl.ds(wid * _NPW, _NPW)], idx_v)

    iota16 = lax.iota(jnp.int32, 16)
    dhi0 = iota16 // 8
    dhi1 = dhi0 + 2
    dlo_c = iota16 - dhi0 * 8
    zeros16 = jnp.zeros((16,), jnp.int32)

    def gather(i, rows, sem):
        return pltpu.async_copy(
            table_hbm.at[idx_v.at[pl.ds(i * _TR, _TR)]], rows, sem
        )

    def wait_gather(rows, sem):
        pltpu.make_async_copy(table_hbm.at[pl.ds(0, _TR)], rows, sem).wait()

    def transpose(rows, trans):
        @plsc.parallel_loop(0, _TR, unroll=8)
        def _(k):
            btl = k // _BB
            jcol = k - btl * _BB
            bv = zeros16 + btl
            jv = zeros16 + jcol
            vec0 = rows[k, pl.ds(0, 16)]
            vec1 = rows[k, pl.ds(16, 16)]
            plsc.store_scatter(trans, [bv, dhi0, dlo_c, jv], vec0)
            plsc.store_scatter(trans, [bv, dhi1, dlo_c, jv], vec1)

    def writes(i, trans, wsem):
        t = base_t + i
        lp = t // (_NBT // _TB)
        bt0 = (t % (_NBT // _TB)) * _TB
        for btl in range(_TB):
            src = trans.at[btl, :, :, pl.ds(0, _BB)]
            pltpu.async_copy(src, out_hbm.at[2 * lp, :, bt0 + btl], wsem)
            pltpu.async_copy(src, out_hbm.at[2 * lp + 1, :, bt0 + btl], wsem)

    def wait_writes(trans, wsem):
        for _ in range(2 * _TB):
            pltpu.make_async_copy(
                trans.at[0, :, :, pl.ds(0, _BB)], out_hbm.at[0, :, 0], wsem
            ).wait()

    gather(0, rows_a, sem_a)
    gather(1, rows_b, sem_b)

    def body(j, carry):
        t0 = 2 * j
        wait_gather(rows_a, sem_a)

        @pl.when(j > 0)
        def _():
            wait_writes(trans_a, wsem_a)

        transpose(rows_a, trans_a)

        @pl.when(t0 + 2 < _TPW)
        def _():
            gather(t0 + 2, rows_a, sem_a)

        writes(t0, trans_a, wsem_a)

        wait_gather(rows_b, sem_b)

        @pl.when(j > 0)
        def _():
            wait_writes(trans_b, wsem_b)

        transpose(rows_b, trans_b)

        @pl.when(t0 + 3 < _TPW)
        def _():
            gather(t0 + 3, rows_b, sem_b)

        writes(t0 + 1, trans_b, wsem_b)
        return carry

    lax.fori_loop(0, _TPW // 2, body, 0)

    if _TPW % 2 == 1:
        wait_gather(rows_a, sem_a)
        wait_writes(trans_a, wsem_a)
        transpose(rows_a, trans_a)
        writes(_TPW - 1, trans_a, wsem_a)
    wait_writes(trans_b, wsem_b)
    wait_writes(trans_a, wsem_a)


_mesh = plsc.VectorSubcoreMesh(
    core_axis_name="c", subcore_axis_name="s", num_cores=_NC, num_subcores=_NS
)

_sc_call = pl.kernel(
    _sc_body,
    out_type=jax.ShapeDtypeStruct(
        (_L * _REPEATS, _D // 8, _NBT, 8, _BB), jnp.float32
    ),
    mesh=_mesh,
    scratch_types=[
        pltpu.VMEM((_NPW,), jnp.int32),
        pltpu.VMEM((_TR, _D), jnp.float32),
        pltpu.VMEM((_TR, _D), jnp.float32),
        pltpu.VMEM((_TB, _D // 8, 8, _BB + 1), jnp.float32),
        pltpu.VMEM((_TB, _D // 8, 8, _BB + 1), jnp.float32),
        pltpu.SemaphoreType.DMA,
        pltpu.SemaphoreType.DMA,
        pltpu.SemaphoreType.DMA,
        pltpu.SemaphoreType.DMA,
    ],
    compiler_params=pltpu.CompilerParams(
        use_tc_tiling_on_sc=False, needs_layout_passes=False
    ),
)


@jax.jit
def kernel(code, table):
    ct_flat = code.T.reshape(-1).astype(jnp.int32)
    out5 = _sc_call(ct_flat, table)
    return out5.transpose((2, 4, 0, 1, 3)).reshape(_B, _L * _REPEATS, _D)
